# baseline (speedup 1.0000x reference)
import jax
import jax.numpy as jnp
from jax import lax
from jax.experimental import pallas as pl
from jax.experimental.pallas import tpu as pltpu
from jax.experimental.pallas import tpu_sc as plsc

N = 10000
E = 320000
D = 128
G = 64
V = 100

NPAD = 10240
VP = 128
BN = 2048
NB = NPAD // BN
NTILES = 32
EPAD = 327680
PAIR = EPAD // 16
EPT0 = 16384
CH = 4096
NCH0 = EPT0 // CH
NCH1 = (PAIR - EPT0) // CH
ZB = 5120
CHUNK = NPAD * VP // 16


def _sc_histogram_body(numbers_hbm, ei_hbm, out_hbm,
                       src_v, dst_v, k_v, idx_v, val_v, zbuf,
                       src_v2, dst_v2, idx_v2,
                       num_sp, c_sp, sem, semz, semg, sems):
    c = lax.axis_index("c")
    s = lax.axis_index("s")
    off = s * CHUNK

    one = jnp.full((16,), 1.0, jnp.float32)
    zero = jnp.zeros((16,), jnp.float32)

    def zrow(i, carry):
        zbuf[pl.ds(i * 16, 16)] = zero
        return carry

    lax.fori_loop(0, ZB // 16, zrow, 0)
    zcopies = [pltpu.async_copy(zbuf, c_sp.at[pl.ds(off + k * ZB, ZB)], semz)
               for k in range(CHUNK // ZB)]

    nslice = NPAD // 16
    pltpu.sync_copy(numbers_hbm.at[pl.ds(s * nslice, nslice)],
                    num_sp.at[pl.ds(s * nslice, nslice)])

    def orow(i, carry):
        val_v[pl.ds(i * 16, 16)] = one
        return carry

    lax.fori_loop(0, CH // 16, orow, 0)

    for zc in zcopies:
        zc.wait()
    plsc.subcore_barrier()

    src_b = [src_v, src_v2]
    dst_b = [dst_v, dst_v2]
    idx_b = [idx_v, idx_v2]

    def pipeline(base, n):
        def fire_load(q, b):
            return (pltpu.async_copy(ei_hbm.at[0, pl.ds(base + q * CH, CH)],
                                     src_b[b], sem),
                    pltpu.async_copy(ei_hbm.at[1, pl.ds(base + q * CH, CH)],
                                     dst_b[b], sem))

        loads = fire_load(0, 0)
        pending = [None, None]
        for q in range(n):
            b = q % 2
            for d in loads:
                d.wait()
            if q + 1 < n:
                loads = fire_load(q + 1, 1 - b)
            if pending[b] is not None:
                pending[b].wait()
            pltpu.async_copy(num_sp.at[src_b[b]], k_v, semg).wait()

            def row(j, carry2):
                r = j * 16
                d16 = dst_b[b][pl.ds(r, 16)]
                k16 = k_v[pl.ds(r, 16)]
                idx_b[b][pl.ds(r, 16)] = d16 * VP + k16
                return carry2

            lax.fori_loop(0, CH // 16, row, 0)

            pending[b] = pltpu.async_copy(val_v, c_sp.at[idx_b[b]], sems,
                                          add=True)
        for d in pending:
            if d is not None:
                d.wait()

    @pl.when(c == 0)
    def _core0():
        pipeline(s * PAIR, NCH0)

    @pl.when(c == 1)
    def _core1():
        pipeline(s * PAIR + EPT0, NCH1)

    plsc.subcore_barrier()

    pltpu.sync_copy(c_sp.at[pl.ds(off, CHUNK)],
                    out_hbm.at[pl.ds(c * (NPAD * VP) + off, CHUNK)])


def _make_sc_histogram():
    return pl.kernel(
        _sc_histogram_body,
        out_type=jax.ShapeDtypeStruct((2 * NPAD * VP,), jnp.float32),
        mesh=plsc.VectorSubcoreMesh(core_axis_name="c", subcore_axis_name="s"),
        compiler_params=pltpu.CompilerParams(needs_layout_passes=False),
        scratch_types=[
            pltpu.VMEM((CH,), jnp.int32),
            pltpu.VMEM((CH,), jnp.int32),
            pltpu.VMEM((CH,), jnp.int32),
            pltpu.VMEM((CH,), jnp.int32),
            pltpu.VMEM((CH,), jnp.float32),
            pltpu.VMEM((ZB,), jnp.float32),
            pltpu.VMEM((CH,), jnp.int32),
            pltpu.VMEM((CH,), jnp.int32),
            pltpu.VMEM((CH,), jnp.int32),
            pltpu.VMEM_SHARED((NPAD,), jnp.int32),
            pltpu.VMEM_SHARED((NPAD * VP,), jnp.float32),
            pltpu.SemaphoreType.DMA,
            pltpu.SemaphoreType.DMA,
            pltpu.SemaphoreType.DMA,
            pltpu.SemaphoreType.DMA,
        ],
    )


def _tc_body(cpair_hbm, num_ref, bat_ref, emb_ref, wm_ref, ws_ref, wn_ref,
             b_ref, out_ref, cb0, cb1, sem0, sem1):
    prec = lax.Precision.HIGHEST
    dot = lambda a, b, da, db, p=prec: lax.dot_general(
        a, b, (((da,), (db,)), ((), ())), precision=p)

    bufs = [cb0, cb1]
    sems = [sem0, sem1]

    def copy(i):
        return pltpu.make_async_copy(
            cpair_hbm.at[:, pl.ds(i * BN, BN), :], bufs[i % 2], sems[i % 2])

    copy(0).start()
    if NB > 1:
        copy(1).start()

    e_self = dot(emb_ref[...], ws_ref[...], 1, 0)
    w_cmb = dot(wm_ref[...], wn_ref[...], 1, 0)
    e_mn = dot(emb_ref[...], w_cmb, 1, 0)

    hi = lax.Precision.DEFAULT
    acc_sum = jnp.zeros((G, D), jnp.float32)
    acc_cnt = jnp.zeros((G, 1), jnp.float32)
    vio = lax.broadcasted_iota(jnp.int32, (VP, 1), 0)
    gio = lax.broadcasted_iota(jnp.int32, (G, 1), 0)

    for i in range(NB):
        copy(i).wait()
        buf = bufs[i % 2]
        cb = buf[0] + buf[1]
        if i + 2 < NB:
            copy(i + 2).start()

        deg = jnp.sum(cb, axis=1, keepdims=True)
        p = cb / jnp.maximum(deg, 1.0)
        agg = dot(p, e_mn, 1, 0, hi)

        nums = num_ref[i]
        onehot_t = (vio == nums).astype(jnp.float32)
        xs = dot(onehot_t, e_self, 0, 0, hi)

        h = jnp.maximum(xs + agg + b_ref[0:1, :], 0.0)

        bat = bat_ref[i]
        oh_g = (gio == bat).astype(jnp.float32)
        acc_sum = acc_sum + dot(oh_g, h, 1, 0, hi)
        acc_cnt = acc_cnt + jnp.sum(oh_g, axis=1, keepdims=True)

    out_ref[...] = acc_sum / jnp.maximum(acc_cnt, 1.0)


def _tc_readout(cpair, numbers3, batch3, emb_p, wm, ws, wn, b2):
    vspec = pl.BlockSpec(memory_space=pltpu.MemorySpace.VMEM)
    return pl.pallas_call(
        _tc_body,
        in_specs=[
            pl.BlockSpec(memory_space=pl.ANY),
            vspec, vspec, vspec, vspec, vspec, vspec, vspec,
        ],
        out_specs=vspec,
        out_shape=jax.ShapeDtypeStruct((G, D), jnp.float32),
        scratch_shapes=[pltpu.VMEM((2, BN, VP), jnp.float32),
                        pltpu.VMEM((2, BN, VP), jnp.float32),
                        pltpu.SemaphoreType.DMA,
                        pltpu.SemaphoreType.DMA],
    )(cpair, numbers3, batch3, emb_p, wm, ws, wn, b2)


def kernel(numbers, edge_index, batch, emb_table, W_msg, W_self, W_nbr, b):
    ei_p = jnp.pad(edge_index, ((0, 0), (0, EPAD - E)), constant_values=N)
    numbers_p = jnp.pad(numbers, (0, NPAD - N))
    cflat = _make_sc_histogram()(numbers_p, ei_p)
    cpair = cflat.reshape(2, NPAD, VP)

    numbers3 = numbers_p.reshape(NB, 1, BN)
    batch3 = jnp.pad(batch, (0, NPAD - N),
                     constant_values=G).reshape(NB, 1, BN)
    emb_p = jnp.pad(emb_table, ((0, VP - V), (0, 0)))
    b2 = jnp.broadcast_to(b[None, :], (8, D))
    return _tc_readout(cpair, numbers3, batch3, emb_p, W_msg, W_self, W_nbr, b2)


# speedup vs baseline: 1.1008x; 1.1008x over previous
import jax
import jax.numpy as jnp
from jax import lax
from jax.experimental import pallas as pl
from jax.experimental.pallas import tpu as pltpu
from jax.experimental.pallas import tpu_sc as plsc

N = 10000
E = 320000
D = 128
G = 64
V = 100

NPAD = 10240
VP = 128
BN = 2048
NB = NPAD // BN
NTILES = 32
EPAD = 327680
PAIR = EPAD // 16
EPT0 = 12288
CH = 2048
NCH0 = EPT0 // CH
NCH1 = (PAIR - EPT0) // CH
ZB = 5120
CHUNK = NPAD * VP // 16


def _sc_histogram_body(numbers_hbm, ei_hbm, out_hbm,
                       src_v, dst_v, k_v, idx_v, val_v, zbuf,
                       src_v2, dst_v2, idx_v2,
                       num_sp, c_sp, sem, semz, semg, sems):
    c = lax.axis_index("c")
    s = lax.axis_index("s")
    off = s * CHUNK

    one = jnp.full((16,), 1.0, jnp.float32)
    zero = jnp.zeros((16,), jnp.float32)

    def zrow(i, carry):
        zbuf[pl.ds(i * 16, 16)] = zero
        return carry

    lax.fori_loop(0, ZB // 16, zrow, 0)
    zcopies = [pltpu.async_copy(zbuf, c_sp.at[pl.ds(off + k * ZB, ZB)], semz)
               for k in range(CHUNK // ZB)]

    nslice = NPAD // 16
    pltpu.sync_copy(numbers_hbm.at[pl.ds(s * nslice, nslice)],
                    num_sp.at[pl.ds(s * nslice, nslice)])

    def orow(i, carry):
        val_v[pl.ds(i * 16, 16)] = one
        return carry

    lax.fori_loop(0, CH // 16, orow, 0)

    for zc in zcopies:
        zc.wait()
    plsc.subcore_barrier()

    src_b = [src_v, src_v2]
    dst_b = [dst_v, dst_v2]
    idx_b = [idx_v, idx_v2]

    def pipeline(base, n):
        def fire_load(q, b):
            return (pltpu.async_copy(ei_hbm.at[0, pl.ds(base + q * CH, CH)],
                                     src_b[b], sem),
                    pltpu.async_copy(ei_hbm.at[1, pl.ds(base + q * CH, CH)],
                                     dst_b[b], sem))

        loads = fire_load(0, 0)
        pending = [None, None]
        for q in range(n):
            b = q % 2
            for d in loads:
                d.wait()
            if q + 1 < n:
                loads = fire_load(q + 1, 1 - b)
            if pending[b] is not None:
                pending[b].wait()
            pltpu.async_copy(num_sp.at[src_b[b]], k_v, semg).wait()

            def row(j, carry2):
                r = j * 16
                d16 = dst_b[b][pl.ds(r, 16)]
                k16 = k_v[pl.ds(r, 16)]
                idx_b[b][pl.ds(r, 16)] = d16 * VP + k16
                return carry2

            lax.fori_loop(0, CH // 16, row, 0)

            pending[b] = pltpu.async_copy(val_v, c_sp.at[idx_b[b]], sems,
                                          add=True)
        for d in pending:
            if d is not None:
                d.wait()

    @pl.when(c == 0)
    def _core0():
        pipeline(s * PAIR, NCH0)

    @pl.when(c == 1)
    def _core1():
        pipeline(s * PAIR + EPT0, NCH1)

    plsc.subcore_barrier()

    pltpu.sync_copy(c_sp.at[pl.ds(off, CHUNK)],
                    out_hbm.at[pl.ds(c * (NPAD * VP) + off, CHUNK)])


def _make_sc_histogram():
    return pl.kernel(
        _sc_histogram_body,
        out_type=jax.ShapeDtypeStruct((2 * NPAD * VP,), jnp.float32),
        mesh=plsc.VectorSubcoreMesh(core_axis_name="c", subcore_axis_name="s"),
        compiler_params=pltpu.CompilerParams(needs_layout_passes=False),
        scratch_types=[
            pltpu.VMEM((CH,), jnp.int32),
            pltpu.VMEM((CH,), jnp.int32),
            pltpu.VMEM((CH,), jnp.int32),
            pltpu.VMEM((CH,), jnp.int32),
            pltpu.VMEM((CH,), jnp.float32),
            pltpu.VMEM((ZB,), jnp.float32),
            pltpu.VMEM((CH,), jnp.int32),
            pltpu.VMEM((CH,), jnp.int32),
            pltpu.VMEM((CH,), jnp.int32),
            pltpu.VMEM_SHARED((NPAD,), jnp.int32),
            pltpu.VMEM_SHARED((NPAD * VP,), jnp.float32),
            pltpu.SemaphoreType.DMA,
            pltpu.SemaphoreType.DMA,
            pltpu.SemaphoreType.DMA,
            pltpu.SemaphoreType.DMA,
        ],
    )


def _tc_body(cpair_hbm, num_ref, bat_ref, emb_ref, wm_ref, ws_ref, wn_ref,
             b_ref, out_ref, cb0, cb1, sem0, sem1):
    prec = lax.Precision.HIGHEST
    dot = lambda a, b, da, db, p=prec: lax.dot_general(
        a, b, (((da,), (db,)), ((), ())), precision=p)

    bufs = [cb0, cb1]
    sems = [sem0, sem1]

    def copy(i):
        return pltpu.make_async_copy(
            cpair_hbm.at[:, pl.ds(i * BN, BN), :], bufs[i % 2], sems[i % 2])

    copy(0).start()
    if NB > 1:
        copy(1).start()

    e_self = dot(emb_ref[...], ws_ref[...], 1, 0)
    w_cmb = dot(wm_ref[...], wn_ref[...], 1, 0)
    e_mn = dot(emb_ref[...], w_cmb, 1, 0)

    hi = lax.Precision.DEFAULT
    acc_sum = jnp.zeros((G, D), jnp.float32)
    acc_cnt = jnp.zeros((G, 1), jnp.float32)
    vio = lax.broadcasted_iota(jnp.int32, (VP, 1), 0)
    gio = lax.broadcasted_iota(jnp.int32, (G, 1), 0)

    for i in range(NB):
        copy(i).wait()
        buf = bufs[i % 2]
        cb = buf[0] + buf[1]
        if i + 2 < NB:
            copy(i + 2).start()

        deg = jnp.sum(cb, axis=1, keepdims=True)
        p = cb / jnp.maximum(deg, 1.0)
        agg = dot(p, e_mn, 1, 0, hi)

        nums = num_ref[i]
        onehot_t = (vio == nums).astype(jnp.float32)
        xs = dot(onehot_t, e_self, 0, 0, hi)

        h = jnp.maximum(xs + agg + b_ref[0:1, :], 0.0)

        bat = bat_ref[i]
        oh_g = (gio == bat).astype(jnp.float32)
        acc_sum = acc_sum + dot(oh_g, h, 1, 0, hi)
        acc_cnt = acc_cnt + jnp.sum(oh_g, axis=1, keepdims=True)

    out_ref[...] = acc_sum / jnp.maximum(acc_cnt, 1.0)


def _tc_readout(cpair, numbers3, batch3, emb_p, wm, ws, wn, b2):
    vspec = pl.BlockSpec(memory_space=pltpu.MemorySpace.VMEM)
    return pl.pallas_call(
        _tc_body,
        in_specs=[
            pl.BlockSpec(memory_space=pl.ANY),
            vspec, vspec, vspec, vspec, vspec, vspec, vspec,
        ],
        out_specs=vspec,
        out_shape=jax.ShapeDtypeStruct((G, D), jnp.float32),
        scratch_shapes=[pltpu.VMEM((2, BN, VP), jnp.float32),
                        pltpu.VMEM((2, BN, VP), jnp.float32),
                        pltpu.SemaphoreType.DMA,
                        pltpu.SemaphoreType.DMA],
    )(cpair, numbers3, batch3, emb_p, wm, ws, wn, b2)


def kernel(numbers, edge_index, batch, emb_table, W_msg, W_self, W_nbr, b):
    ei_p = jnp.pad(edge_index, ((0, 0), (0, EPAD - E)), constant_values=N)
    numbers_p = jnp.pad(numbers, (0, NPAD - N))
    cflat = _make_sc_histogram()(numbers_p, ei_p)
    cpair = cflat.reshape(2, NPAD, VP)

    numbers3 = numbers_p.reshape(NB, 1, BN)
    batch3 = jnp.pad(batch, (0, NPAD - N),
                     constant_values=G).reshape(NB, 1, BN)
    emb_p = jnp.pad(emb_table, ((0, VP - V), (0, 0)))
    b2 = jnp.broadcast_to(b[None, :], (8, D))
    return _tc_readout(cpair, numbers3, batch3, emb_p, W_msg, W_self, W_nbr, b2)


# speedup vs baseline: 1.1238x; 1.0209x over previous
import jax
import jax.numpy as jnp
from jax import lax
from jax.experimental import pallas as pl
from jax.experimental.pallas import tpu as pltpu
from jax.experimental.pallas import tpu_sc as plsc

N = 10000
E = 320000
D = 128
G = 64
V = 100

NPAD = 10240
VP = 128
BN = 2048
NB = NPAD // BN
NTILES = 32
EPAD = 327680
PAIR = EPAD // 16
EPT0 = 12288
CH = 1024
NCH0 = EPT0 // CH
NCH1 = (PAIR - EPT0) // CH
ZB = 5120
CHUNK = NPAD * VP // 16


def _sc_histogram_body(numbers_hbm, ei_hbm, out_hbm,
                       src_v, dst_v, k_v, idx_v, val_v, zbuf,
                       src_v2, dst_v2, idx_v2,
                       num_sp, c_sp, sem, semz, semg, sems):
    c = lax.axis_index("c")
    s = lax.axis_index("s")
    off = s * CHUNK

    one = jnp.full((16,), 1.0, jnp.float32)
    zero = jnp.zeros((16,), jnp.float32)

    def zrow(i, carry):
        zbuf[pl.ds(i * 16, 16)] = zero
        return carry

    lax.fori_loop(0, ZB // 16, zrow, 0)
    zcopies = [pltpu.async_copy(zbuf, c_sp.at[pl.ds(off + k * ZB, ZB)], semz)
               for k in range(CHUNK // ZB)]

    nslice = NPAD // 16
    pltpu.sync_copy(numbers_hbm.at[pl.ds(s * nslice, nslice)],
                    num_sp.at[pl.ds(s * nslice, nslice)])

    def orow(i, carry):
        val_v[pl.ds(i * 16, 16)] = one
        return carry

    lax.fori_loop(0, CH // 16, orow, 0)

    for zc in zcopies:
        zc.wait()
    plsc.subcore_barrier()

    src_b = [src_v, src_v2]
    dst_b = [dst_v, dst_v2]
    idx_b = [idx_v, idx_v2]

    def pipeline(base, n):
        def fire_load(q, b):
            return (pltpu.async_copy(ei_hbm.at[0, pl.ds(base + q * CH, CH)],
                                     src_b[b], sem),
                    pltpu.async_copy(ei_hbm.at[1, pl.ds(base + q * CH, CH)],
                                     dst_b[b], sem))

        loads = fire_load(0, 0)
        pending = [None, None]
        for q in range(n):
            b = q % 2
            for d in loads:
                d.wait()
            if q + 1 < n:
                loads = fire_load(q + 1, 1 - b)
            if pending[b] is not None:
                pending[b].wait()
            pltpu.async_copy(num_sp.at[src_b[b]], k_v, semg).wait()

            def row(j, carry2):
                r = j * 16
                d16 = dst_b[b][pl.ds(r, 16)]
                k16 = k_v[pl.ds(r, 16)]
                idx_b[b][pl.ds(r, 16)] = d16 * VP + k16
                return carry2

            lax.fori_loop(0, CH // 16, row, 0)

            pending[b] = pltpu.async_copy(val_v, c_sp.at[idx_b[b]], sems,
                                          add=True)
        for d in pending:
            if d is not None:
                d.wait()

    @pl.when(c == 0)
    def _core0():
        pipeline(s * PAIR, NCH0)

    @pl.when(c == 1)
    def _core1():
        pipeline(s * PAIR + EPT0, NCH1)

    plsc.subcore_barrier()

    pltpu.sync_copy(c_sp.at[pl.ds(off, CHUNK)],
                    out_hbm.at[pl.ds(c * (NPAD * VP) + off, CHUNK)])


def _make_sc_histogram():
    return pl.kernel(
        _sc_histogram_body,
        out_type=jax.ShapeDtypeStruct((2 * NPAD * VP,), jnp.float32),
        mesh=plsc.VectorSubcoreMesh(core_axis_name="c", subcore_axis_name="s"),
        compiler_params=pltpu.CompilerParams(needs_layout_passes=False),
        scratch_types=[
            pltpu.VMEM((CH,), jnp.int32),
            pltpu.VMEM((CH,), jnp.int32),
            pltpu.VMEM((CH,), jnp.int32),
            pltpu.VMEM((CH,), jnp.int32),
            pltpu.VMEM((CH,), jnp.float32),
            pltpu.VMEM((ZB,), jnp.float32),
            pltpu.VMEM((CH,), jnp.int32),
            pltpu.VMEM((CH,), jnp.int32),
            pltpu.VMEM((CH,), jnp.int32),
            pltpu.VMEM_SHARED((NPAD,), jnp.int32),
            pltpu.VMEM_SHARED((NPAD * VP,), jnp.float32),
            pltpu.SemaphoreType.DMA,
            pltpu.SemaphoreType.DMA,
            pltpu.SemaphoreType.DMA,
            pltpu.SemaphoreType.DMA,
        ],
    )


def _tc_body(cpair_hbm, num_ref, bat_ref, emb_ref, wm_ref, ws_ref, wn_ref,
             b_ref, out_ref, cb0, cb1, sem0, sem1):
    prec = lax.Precision.HIGHEST
    dot = lambda a, b, da, db, p=prec: lax.dot_general(
        a, b, (((da,), (db,)), ((), ())), precision=p)

    bufs = [cb0, cb1]
    sems = [sem0, sem1]

    def copy(i):
        return pltpu.make_async_copy(
            cpair_hbm.at[:, pl.ds(i * BN, BN), :], bufs[i % 2], sems[i % 2])

    copy(0).start()
    if NB > 1:
        copy(1).start()

    e_self = dot(emb_ref[...], ws_ref[...], 1, 0)
    w_cmb = dot(wm_ref[...], wn_ref[...], 1, 0)
    e_mn = dot(emb_ref[...], w_cmb, 1, 0)

    hi = lax.Precision.DEFAULT
    acc_sum = jnp.zeros((G, D), jnp.float32)
    acc_cnt = jnp.zeros((G, 1), jnp.float32)
    vio = lax.broadcasted_iota(jnp.int32, (VP, 1), 0)
    gio = lax.broadcasted_iota(jnp.int32, (G, 1), 0)

    for i in range(NB):
        copy(i).wait()
        buf = bufs[i % 2]
        cb = buf[0] + buf[1]
        if i + 2 < NB:
            copy(i + 2).start()

        deg = jnp.sum(cb, axis=1, keepdims=True)
        p = cb / jnp.maximum(deg, 1.0)
        agg = dot(p, e_mn, 1, 0, hi)

        nums = num_ref[i]
        onehot_t = (vio == nums).astype(jnp.float32)
        xs = dot(onehot_t, e_self, 0, 0, hi)

        h = jnp.maximum(xs + agg + b_ref[0:1, :], 0.0)

        bat = bat_ref[i]
        oh_g = (gio == bat).astype(jnp.float32)
        acc_sum = acc_sum + dot(oh_g, h, 1, 0, hi)
        acc_cnt = acc_cnt + jnp.sum(oh_g, axis=1, keepdims=True)

    out_ref[...] = acc_sum / jnp.maximum(acc_cnt, 1.0)


def _tc_readout(cpair, numbers3, batch3, emb_p, wm, ws, wn, b2):
    vspec = pl.BlockSpec(memory_space=pltpu.MemorySpace.VMEM)
    return pl.pallas_call(
        _tc_body,
        in_specs=[
            pl.BlockSpec(memory_space=pl.ANY),
            vspec, vspec, vspec, vspec, vspec, vspec, vspec,
        ],
        out_specs=vspec,
        out_shape=jax.ShapeDtypeStruct((G, D), jnp.float32),
        scratch_shapes=[pltpu.VMEM((2, BN, VP), jnp.float32),
                        pltpu.VMEM((2, BN, VP), jnp.float32),
                        pltpu.SemaphoreType.DMA,
                        pltpu.SemaphoreType.DMA],
    )(cpair, numbers3, batch3, emb_p, wm, ws, wn, b2)


def kernel(numbers, edge_index, batch, emb_table, W_msg, W_self, W_nbr, b):
    ei_p = jnp.pad(edge_index, ((0, 0), (0, EPAD - E)), constant_values=N)
    numbers_p = jnp.pad(numbers, (0, NPAD - N))
    cflat = _make_sc_histogram()(numbers_p, ei_p)
    cpair = cflat.reshape(2, NPAD, VP)

    numbers3 = numbers_p.reshape(NB, 1, BN)
    batch3 = jnp.pad(batch, (0, NPAD - N),
                     constant_values=G).reshape(NB, 1, BN)
    emb_p = jnp.pad(emb_table, ((0, VP - V), (0, 0)))
    b2 = jnp.broadcast_to(b[None, :], (8, D))
    return _tc_readout(cpair, numbers3, batch3, emb_p, W_msg, W_self, W_nbr, b2)
